# Initial kernel scaffold; baseline (speedup 1.0000x reference)
import jax
import jax.numpy as jnp
from jax.experimental import pallas as pl


def kernel(inp, indices, weights, tau_syn_weights):
    raise NotImplementedError("write your pallas kernel here")


# speedup vs baseline: 3.3474x; 3.3474x over previous
import functools

import jax
import jax.numpy as jnp
from jax import lax
from jax.experimental import pallas as pl
from jax.experimental.pallas import tpu as pltpu, tpu_sc as plsc

N_NEURONS = 10000
N_BKG = 100
NNZ = 160000
N_SYN = 5
T = 256
BKG_RATE = 250

NC, NS = 2, 16
NW = NC * NS
K = 160
NPL = K // 16
N_PAD = 10240
NCHUNK = N_PAD // K
CHUNKS_PER_W = (NCHUNK + NW - 1) // NW
BLKN = 4096
NNZ_PAD = NNZ + BLKN
RP_LEN = N_PAD + 8
WL = K * N_SYN * N_BKG


def _sc_scatter_body(rows_hbm, cols_hbm, w_hbm, tau_hbm, rp_hbm, zeros_hbm,
                     out_hbm, rows_v, cols_v, w_v,
                     t0_v, t1_v, t2_v, t3_v, t4_v, rp_v, wl_v):
    tau_vs = (t0_v, t1_v, t2_v, t3_v, t4_v)
    wid = lax.axis_index("s") * NC + lax.axis_index("c")

    lane = lax.iota(jnp.int32, 16)

    for it in range(CHUNKS_PER_W):
        chunk = wid + NW * it

        @pl.when(chunk < NCHUNK)
        def _process():
            n0 = pl.multiple_of(chunk * K, 8)
            pltpu.sync_copy(rp_hbm.at[pl.ds(n0, K + 8)], rp_v)
            a = plsc.load_gather(rp_v, [lane * NPL])
            b = plsc.load_gather(rp_v, [lane * NPL + NPL])
            p1 = jnp.max(b)

            pltpu.sync_copy(zeros_hbm, wl_v)

            ws0 = jnp.min(a) & ~jnp.int32(7)

            def window(ws_carry):
                ws = pl.multiple_of(ws_carry, 8)
                we = ws + BLKN
                pltpu.sync_copy(rows_hbm.at[pl.ds(ws, BLKN)], rows_v)
                pltpu.sync_copy(cols_hbm.at[pl.ds(ws, BLKN)], cols_v)
                pltpu.sync_copy(w_hbm.at[pl.ds(ws, BLKN)], w_v)
                for s in range(N_SYN):
                    pltpu.sync_copy(
                        tau_hbm.at[pl.ds(
                            pl.multiple_of(s * NNZ_PAD + ws, 8), BLKN)],
                        tau_vs[s])
                c0 = jnp.maximum(a, ws)
                bmin = jnp.minimum(b, we)
                steps = jnp.max(jnp.maximum(bmin - c0, 0))

                def step(i, _):
                    ci = c0 + i
                    m = ci < bmin
                    off = jnp.minimum(ci - ws, BLKN - 1)
                    r16 = plsc.load_gather(rows_v, [off])
                    c16 = plsc.load_gather(cols_v, [off])
                    w16 = plsc.load_gather(w_v, [off])
                    base = (r16 - n0) * (N_SYN * N_BKG) + c16
                    for s in range(N_SYN):
                        t16 = plsc.load_gather(tau_vs[s], [off])
                        plsc.addupdate_scatter(
                            wl_v, [base + s * N_BKG], w16 * t16, mask=m)
                    return 0

                lax.fori_loop(0, steps, step, 0)
                return ws + BLKN

            lax.while_loop(lambda ws: ws < p1, window, ws0)

            pltpu.sync_copy(
                wl_v, out_hbm.at[pl.ds(pl.multiple_of(chunk * WL, 8), WL)])


_sc_scatter = pl.kernel(
    _sc_scatter_body,
    out_type=jax.ShapeDtypeStruct((NCHUNK * WL,), jnp.float32),
    mesh=plsc.VectorSubcoreMesh(core_axis_name="c", subcore_axis_name="s",
                                num_cores=NC, num_subcores=NS),
    compiler_params=pltpu.CompilerParams(needs_layout_passes=False),
    scratch_types=[
        pltpu.VMEM((BLKN,), jnp.int32),
        pltpu.VMEM((BLKN,), jnp.int32),
        pltpu.VMEM((BLKN,), jnp.float32),
        pltpu.VMEM((BLKN,), jnp.float32),
        pltpu.VMEM((BLKN,), jnp.float32),
        pltpu.VMEM((BLKN,), jnp.float32),
        pltpu.VMEM((BLKN,), jnp.float32),
        pltpu.VMEM((BLKN,), jnp.float32),
        pltpu.VMEM((K + 8,), jnp.int32),
        pltpu.VMEM((WL,), jnp.float32),
    ],
)


def _mm_body(s_ref, w_ref, o_ref):
    o_ref[...] = lax.dot_general(
        s_ref[...], w_ref[...], (((1,), (1,)), ((), ())),
        preferred_element_type=jnp.float32)


_BN = 2048


@jax.jit
def kernel(inp, indices, weights, tau_syn_weights):
    bsz, t, _ = inp.shape
    bt = bsz * t

    spikes = (jax.random.uniform(jax.random.key(42), (bsz, t, N_BKG))
              < BKG_RATE * 0.001).astype(jnp.float32).reshape(bt, N_BKG)

    rows = indices[:, 0].astype(jnp.int32)
    cols = indices[:, 1].astype(jnp.int32)
    w32 = weights.astype(jnp.float32)
    tau_t = tau_syn_weights.astype(jnp.float32).T

    rp = jnp.searchsorted(rows, jnp.arange(RP_LEN, dtype=jnp.int32)
                          ).astype(jnp.int32)

    rows_p = jnp.concatenate(
        [rows, jnp.full((NNZ_PAD - NNZ,), N_PAD, jnp.int32)])
    cols_p = jnp.concatenate([cols, jnp.zeros((NNZ_PAD - NNZ,), jnp.int32)])
    w_p = jnp.concatenate([w32, jnp.zeros((NNZ_PAD - NNZ,), jnp.float32)])
    tau_p = jnp.concatenate(
        [tau_t, jnp.zeros((N_SYN, NNZ_PAD - NNZ), jnp.float32)],
        axis=1).reshape(N_SYN * NNZ_PAD)
    zeros = jnp.zeros((WL,), jnp.float32)

    w_flat = _sc_scatter(rows_p, cols_p, w_p, tau_p, rp, zeros)
    w2 = w_flat.reshape(N_PAD * N_SYN, N_BKG)

    nout = N_NEURONS * N_SYN
    out = pl.pallas_call(
        _mm_body,
        grid=((nout + _BN - 1) // _BN,),
        in_specs=[
            pl.BlockSpec((bt, N_BKG), lambda j: (0, 0)),
            pl.BlockSpec((_BN, N_BKG), lambda j: (j, 0)),
        ],
        out_specs=pl.BlockSpec((bt, _BN), lambda j: (0, j)),
        out_shape=jax.ShapeDtypeStruct((bt, nout), jnp.float32),
    )(spikes, w2)

    return out.reshape(bsz, t, nout)


# speedup vs baseline: 4.6258x; 1.3819x over previous
import jax
import jax.numpy as jnp
from jax import lax
from jax.experimental import pallas as pl
from jax.experimental.pallas import tpu as pltpu, tpu_sc as plsc

N_NEURONS = 10000
N_BKG = 100
NNZ = 160000
N_SYN = 5
BKG_RATE = 250

NC, NS = 2, 16
NW = NC * NS
K = 160
NPL = K // 16
N_PAD = 10240
NCHUNK = N_PAD // K
CHUNKS_PER_W = (NCHUNK + NW - 1) // NW
BLKN = 4096
NB10 = N_PAD // NPL
WL = K * N_SYN * N_BKG


def _sc_scatter_body(idx_hbm, w_hbm, tau_hbm, rp_hbm, zeros_hbm,
                     out_hbm, idx_v, w_v, tau_v, rp_v, wl_v):
    wid = lax.axis_index("s") * NC + lax.axis_index("c")
    lane = lax.iota(jnp.int32, 16)

    for it in range(CHUNKS_PER_W):
        chunk = wid + NW * it
        n0 = chunk * K

        pltpu.sync_copy(
            rp_hbm.at[pl.ds(pl.multiple_of(chunk * 16, 8), 24)], rp_v)
        a = plsc.load_gather(rp_v, [lane])
        b = plsc.load_gather(rp_v, [lane + 1])
        p1 = jnp.max(b)

        pltpu.sync_copy(zeros_hbm, wl_v)

        ws0 = jnp.min(a) & ~jnp.int32(7)

        def window(ws_carry, a=a, b=b, n0=n0):
            ws = pl.multiple_of(ws_carry, 8)
            we = ws + BLKN
            wd = pl.multiple_of(jnp.minimum(ws, NNZ - BLKN), 8)
            pltpu.sync_copy(
                idx_hbm.at[pl.ds(pl.multiple_of(wd * 2, 8), 2 * BLKN)],
                idx_v)
            pltpu.sync_copy(w_hbm.at[pl.ds(wd, BLKN)], w_v)
            pltpu.sync_copy(
                tau_hbm.at[pl.ds(pl.multiple_of(wd * 5, 8), 5 * BLKN)],
                tau_v)
            c0 = jnp.maximum(a, ws)
            bmin = jnp.minimum(b, we)
            steps = jnp.max(jnp.maximum(bmin - c0, 0))

            def step(i, _, c0=c0, bmin=bmin, wd=wd, n0=n0):
                ci = c0 + i
                m = ci < bmin
                off = jnp.minimum(ci - wd, BLKN - 1)
                r16 = plsc.load_gather(idx_v, [off * 2])
                c16 = plsc.load_gather(idx_v, [off * 2 + 1])
                w16 = plsc.load_gather(w_v, [off])
                base = (r16 - n0) * (N_SYN * N_BKG) + c16
                toff = off * N_SYN
                for s in range(N_SYN):
                    t16 = plsc.load_gather(tau_v, [toff + s])
                    plsc.addupdate_scatter(
                        wl_v, [base + s * N_BKG], w16 * t16, mask=m)
                return 0

            lax.fori_loop(0, steps, step, 0)
            return ws + BLKN

        lax.while_loop(lambda ws, p1=p1: ws < p1, window, ws0)

        pltpu.sync_copy(
            wl_v, out_hbm.at[pl.ds(pl.multiple_of(chunk * WL, 8), WL)])


_sc_scatter = pl.kernel(
    _sc_scatter_body,
    out_type=jax.ShapeDtypeStruct((NCHUNK * WL,), jnp.float32),
    mesh=plsc.VectorSubcoreMesh(core_axis_name="c", subcore_axis_name="s",
                                num_cores=NC, num_subcores=NS),
    compiler_params=pltpu.CompilerParams(needs_layout_passes=False),
    scratch_types=[
        pltpu.VMEM((2 * BLKN,), jnp.int32),
        pltpu.VMEM((BLKN,), jnp.float32),
        pltpu.VMEM((N_SYN * BLKN,), jnp.float32),
        pltpu.VMEM((24,), jnp.int32),
        pltpu.VMEM((WL,), jnp.float32),
    ],
)


def _mm_body(s_ref, w_ref, o_ref):
    o_ref[...] = lax.dot_general(
        s_ref[...], w_ref[...], (((1,), (1,)), ((), ())),
        preferred_element_type=jnp.float32)


_BN = 2048


@jax.jit
def kernel(inp, indices, weights, tau_syn_weights):
    bsz, t, _ = inp.shape
    bt = bsz * t

    spikes = (jax.random.uniform(jax.random.key(42), (bsz, t, N_BKG))
              < BKG_RATE * 0.001).astype(jnp.float32).reshape(bt, N_BKG)

    idx_flat = indices.astype(jnp.int32).reshape(2 * NNZ)
    w32 = weights.astype(jnp.float32)
    tau_flat = tau_syn_weights.astype(jnp.float32).reshape(N_SYN * NNZ)

    rows = idx_flat[::2]
    rp = jnp.searchsorted(
        rows, jnp.arange(0, (NB10 + 8) * NPL, NPL, dtype=jnp.int32)
    ).astype(jnp.int32)

    zeros = jnp.zeros((WL,), jnp.float32)

    w_flat = _sc_scatter(idx_flat, w32, tau_flat, rp, zeros)
    w2 = w_flat.reshape(N_PAD * N_SYN, N_BKG)

    nout = N_NEURONS * N_SYN
    out = pl.pallas_call(
        _mm_body,
        grid=((nout + _BN - 1) // _BN,),
        in_specs=[
            pl.BlockSpec((bt, N_BKG), lambda j: (0, 0)),
            pl.BlockSpec((_BN, N_BKG), lambda j: (j, 0)),
        ],
        out_specs=pl.BlockSpec((bt, _BN), lambda j: (0, j)),
        out_shape=jax.ShapeDtypeStruct((bt, nout), jnp.float32),
    )(spikes, w2)

    return out.reshape(bsz, t, nout)


# speedup vs baseline: 7.7318x; 1.6715x over previous
import jax
import jax.numpy as jnp
from jax import lax
from jax.experimental import pallas as pl
from jax.experimental.pallas import tpu as pltpu, tpu_sc as plsc

N_NEURONS = 10000
N_BKG = 100
NNZ = 160000
N_SYN = 5
BKG_RATE = 250

NC, NS = 2, 16
NW = NC * NS
K = 160
NPL = K // 16
N_PAD = 10240
NCHUNK = N_PAD // K
CHUNKS_PER_W = NCHUNK // NW
BLKN = 3072
NBIN = 1040
WROW = K * N_SYN
WCOL = 128
PPT = NNZ // NS
PPW = PPT // 2
ASTEP = (PPW + 15) // 16


def _bin10(r):
    return lax.shift_right_logical(r * 52429, 19)


def _sc_body(idx_hbm, w_hbm, tau_hbm, zeros_hbm, out_hbm, shared):
    cid = lax.axis_index("c")
    sid = lax.axis_index("s")
    wid = sid * NC + cid
    lane = lax.iota(jnp.int32, 16)

    if True:
        def phase_a(idx2_v, rp_loc):
            fullv = jnp.full((16,), NNZ, jnp.int32)
            def init(v, _):
                rp_loc[pl.ds(v * 16, 16)] = fullv
                return 0
            lax.fori_loop(0, NBIN // 16, init, 0)

            for h in range(2):
                dma_off = pl.multiple_of(
                    jnp.maximum(sid * (2 * PPT) + h * (2 * PPW) - 8, 0), 8)
                pltpu.sync_copy(idx_hbm.at[pl.ds(dma_off, 2 * PPW + 8)],
                                idx2_v)
                g0p = lax.shift_right_logical(dma_off, 1)
                pbase = sid * PPT + h * PPW
                pend = pbase + PPW

                def scan(i, _, g0p=g0p, pbase=pbase, pend=pend):
                    p = pbase + i * 16 + lane
                    lpc = jnp.minimum(p - g0p, PPW + 3)
                    r = plsc.load_gather(idx2_v, [2 * lpc])
                    rprev = plsc.load_gather(
                        idx2_v, [jnp.maximum(2 * lpc - 2, 0)])
                    q = _bin10(r)
                    qprev = jnp.where(p == 0, -1, _bin10(rprev))
                    m = (q != qprev) & (p < pend)
                    plsc.store_scatter(rp_loc, [q], p, mask=m)
                    return 0

                lax.fori_loop(0, ASTEP, scan, 0)

            pltpu.sync_copy(
                rp_loc,
                shared.at[pl.ds(pl.multiple_of(sid * NBIN, 8), NBIN)])

        pl.run_scoped(phase_a,
                      pltpu.VMEM((2 * PPW + 8,), jnp.int32),
                      pltpu.VMEM((NBIN,), jnp.int32))
        plsc.subcore_barrier()

        @pl.when(sid == 0)
        def _reduce():
            def phase_b(stage_v, fin_v):
                pltpu.sync_copy(shared, stage_v)

                def redv(v, _):
                    acc = stage_v[pl.ds(v * 16, 16)]
                    for t in range(1, NS):
                        acc = jnp.minimum(
                            acc, stage_v[pl.ds(t * NBIN + v * 16, 16)])
                    fin_v[pl.ds(v * 16, 16)] = acc
                    return 0
                lax.fori_loop(0, NBIN // 16, redv, 0)

                def sufv(i, carry):
                    v = NBIN // 16 - 1 - i
                    seg = fin_v[pl.ds(v * 16, 16)]
                    rs = lax.rev(seg, (0,))
                    run = -plsc.cummax(-rs)
                    comb = jnp.minimum(run, carry)
                    fin_v[pl.ds(v * 16, 16)] = lax.rev(comb, (0,))
                    return jnp.min(comb)
                lax.fori_loop(0, NBIN // 16, sufv, jnp.int32(NNZ))

                pltpu.sync_copy(fin_v, shared.at[pl.ds(0, NBIN)])

            pl.run_scoped(phase_b,
                          pltpu.VMEM((NS * NBIN,), jnp.int32),
                          pltpu.VMEM((NBIN,), jnp.int32))

        plsc.subcore_barrier()

        def phase_c(idx_v, w_v, tau_v, rp_v, wl_v):
            for it in range(CHUNKS_PER_W):
                chunk = wid + NW * it
                n0 = chunk * K

                pltpu.sync_copy(
                    shared.at[pl.ds(pl.multiple_of(chunk * 16, 8), 24)],
                    rp_v)
                a = plsc.load_gather(rp_v, [lane])
                b = plsc.load_gather(rp_v, [lane + 1])
                p1 = jnp.max(b)

                pltpu.sync_copy(zeros_hbm, wl_v)

                ws0 = jnp.min(a) & ~jnp.int32(7)

                def window(ws_carry, a=a, b=b, n0=n0):
                    ws = pl.multiple_of(ws_carry, 8)
                    we = ws + BLKN
                    wd = pl.multiple_of(jnp.minimum(ws, NNZ - BLKN), 8)
                    pltpu.sync_copy(
                        idx_hbm.at[pl.ds(pl.multiple_of(wd * 2, 8),
                                         2 * BLKN)], idx_v)
                    pltpu.sync_copy(w_hbm.at[pl.ds(wd, BLKN)], w_v)
                    pltpu.sync_copy(
                        tau_hbm.at[pl.ds(pl.multiple_of(wd * 5, 8),
                                         5 * BLKN)], tau_v)
                    c0 = jnp.maximum(a, ws)
                    bmin = jnp.minimum(b, we)
                    steps = jnp.max(jnp.maximum(bmin - c0, 0))

                    def step(i, _, c0=c0, bmin=bmin, wd=wd, n0=n0):
                        ci = c0 + i
                        m = ci < bmin
                        off = jnp.minimum(ci - wd, BLKN - 1)
                        r16 = plsc.load_gather(idx_v, [off * 2])
                        c16 = plsc.load_gather(idx_v, [off * 2 + 1])
                        w16 = plsc.load_gather(w_v, [off])
                        row = (r16 - n0) * N_SYN
                        toff = off * N_SYN
                        for s in range(N_SYN):
                            t16 = plsc.load_gather(tau_v, [toff + s])
                            plsc.addupdate_scatter(
                                wl_v, [row + s, c16], w16 * t16, mask=m)
                        return 0

                    lax.fori_loop(0, steps, step, 0)
                    return ws + BLKN

                lax.while_loop(lambda ws, p1=p1: ws < p1, window, ws0)

                pltpu.sync_copy(
                    wl_v,
                    out_hbm.at[pl.ds(pl.multiple_of(chunk * WROW, 8), WROW)])

        pl.run_scoped(phase_c,
                      pltpu.VMEM((2 * BLKN,), jnp.int32),
                      pltpu.VMEM((BLKN,), jnp.float32),
                      pltpu.VMEM((N_SYN * BLKN,), jnp.float32),
                      pltpu.VMEM((24,), jnp.int32),
                      pltpu.VMEM((WROW, WCOL), jnp.float32))

_sc_scatter = pl.kernel(
    _sc_body,
    out_type=jax.ShapeDtypeStruct((NCHUNK * WROW, WCOL), jnp.float32),
    mesh=plsc.VectorSubcoreMesh(core_axis_name="c", subcore_axis_name="s",
                                num_cores=NC, num_subcores=NS),
    compiler_params=pltpu.CompilerParams(needs_layout_passes=False),
    scratch_types=[pltpu.VMEM_SHARED((NS * NBIN,), jnp.int32)],
)


def _mm_body(s_ref, w_ref, o_ref):
    o_ref[...] = lax.dot_general(
        s_ref[...], w_ref[...], (((1,), (1,)), ((), ())),
        preferred_element_type=jnp.float32)


_BN = 2048


@jax.jit
def kernel(inp, indices, weights, tau_syn_weights):
    bsz, t, _ = inp.shape
    bt = bsz * t

    spikes = (jax.random.uniform(jax.random.key(42), (bsz, t, N_BKG))
              < BKG_RATE * 0.001).astype(jnp.float32).reshape(bt, N_BKG)
    spikes = jnp.pad(spikes, ((0, 0), (0, WCOL - N_BKG)))

    idx_flat = indices.astype(jnp.int32).reshape(2 * NNZ)
    w32 = weights.astype(jnp.float32)
    tau_flat = tau_syn_weights.astype(jnp.float32).reshape(N_SYN * NNZ)
    zeros = jnp.zeros((WROW, WCOL), jnp.float32)

    w2 = _sc_scatter(idx_flat, w32, tau_flat, zeros)

    nout = N_NEURONS * N_SYN
    out = pl.pallas_call(
        _mm_body,
        grid=((nout + _BN - 1) // _BN,),
        in_specs=[
            pl.BlockSpec((bt, WCOL), lambda j: (0, 0)),
            pl.BlockSpec((_BN, WCOL), lambda j: (j, 0)),
        ],
        out_specs=pl.BlockSpec((bt, _BN), lambda j: (0, j)),
        out_shape=jax.ShapeDtypeStruct((bt, nout), jnp.float32),
    )(spikes, w2)

    return out.reshape(bsz, t, nout)
